# baseline (speedup 1.0000x reference)
import functools

import jax
import jax.numpy as jnp
from jax.experimental import pallas as pl
from jax.experimental.pallas import tpu as pltpu

_NEG_SLOPE = 0.2


def _gat_kernel(x_ref, adj_ref, wt_ref, as_ref, ad_ref, b_ref, out_ref,
                ha_ref, ht_ref, asc_ref, asr_ref, adr_ref, d_ref, acc_ref,
                *, block_j, n_nodes):
    j = pl.program_id(0)
    nsteps = pl.num_programs(0)
    f = wt_ref.shape[0]

    @pl.when(j == 0)
    def _prologue():
        h = jax.lax.dot_general(x_ref[...], wt_ref[...],
                                (((1,), (1,)), ((), ())),
                                preferred_element_type=jnp.float32)
        ha_ref[:, :f] = h
        ha_ref[:, f:] = jnp.ones((n_nodes, 1), jnp.float32)
        ht_ref[...] = jnp.transpose(h)
        a_s_row = jax.lax.dot_general(as_ref[...].reshape(1, f), h,
                                      (((1,), (1,)), ((), ())),
                                      preferred_element_type=jnp.float32)
        asr_ref[...] = a_s_row
        asc_ref[...] = jnp.transpose(a_s_row)
        adr_ref[...] = jax.lax.dot_general(ad_ref[...].reshape(1, f), h,
                                           (((1,), (1,)), ((), ())),
                                           preferred_element_type=jnp.float32)
        d_ref[...] = (
            jax.lax.broadcasted_iota(jnp.int32, (block_j, n_nodes), 1)
            - jax.lax.broadcasted_iota(jnp.int32, (block_j, n_nodes), 0))
        acc_ref[...] = jnp.zeros((f + 1, n_nodes), jnp.float32)

    @pl.when(j > 0)
    def _body():
        jj = j - 1
        a_s_blk = asc_ref[pl.ds(jj * block_j, block_j), :]
        e = a_s_blk + adr_ref[...]
        e = jnp.maximum(e, _NEG_SLOPE * e)
        keep = (adj_ref[...] != 0) & (d_ref[...] != jj * block_j)
        p = jnp.where(keep, jnp.exp(e), 0.0)

        ha_blk = ha_ref[pl.ds(jj * block_j, block_j), :]
        acc_ref[...] += jax.lax.dot_general(ha_blk, p,
                                            (((0,), (0,)), ((), ())),
                                            preferred_element_type=jnp.float32)

    @pl.when(j == nsteps - 1)
    def _epilogue():
        diag = asr_ref[...] + adr_ref[...]
        diag = jnp.maximum(diag, _NEG_SLOPE * diag)
        p_diag = jnp.exp(diag)
        denom = acc_ref[f:, :] + p_diag
        inv = 1.0 / (denom + 1e-16)
        b_col = b_ref[...].reshape(f, 1)
        out_ref[...] = (acc_ref[:f, :] + ht_ref[...] * p_diag) * inv + b_col


def kernel(x, adj, W, att_src, att_dst, bias):
    n, in_f = x.shape
    f = W.shape[1]

    block_j = 512
    grid = (n // block_j + 1,)

    out_t = pl.pallas_call(
        functools.partial(_gat_kernel, block_j=block_j, n_nodes=n),
        grid=grid,
        in_specs=[
            pl.BlockSpec((n, in_f), lambda j: (0, 0)),
            pl.BlockSpec((block_j, n),
                         lambda j: (jnp.maximum(j - 1, 0), 0)),
            pl.BlockSpec((f, in_f), lambda j: (0, 0)),
            pl.BlockSpec((1, 1, f), lambda j: (0, 0, 0)),
            pl.BlockSpec((1, 1, f), lambda j: (0, 0, 0)),
            pl.BlockSpec((f,), lambda j: (0,)),
        ],
        out_specs=pl.BlockSpec((f, n), lambda j: (0, 0)),
        out_shape=jax.ShapeDtypeStruct((f, n), jnp.float32),
        scratch_shapes=[
            pltpu.VMEM((n, f + 1), jnp.float32),
            pltpu.VMEM((f, n), jnp.float32),
            pltpu.VMEM((n, 1), jnp.float32),
            pltpu.VMEM((1, n), jnp.float32),
            pltpu.VMEM((1, n), jnp.float32),
            pltpu.VMEM((block_j, n), jnp.int32),
            pltpu.VMEM((f + 1, n), jnp.float32),
        ],
        compiler_params=pltpu.CompilerParams(
            dimension_semantics=("arbitrary",),
        ),
    )(x, adj, W.T, att_src, att_dst, bias)
    return out_t.T


# speedup vs baseline: 1.0469x; 1.0469x over previous
import functools

import jax
import jax.numpy as jnp
from jax.experimental import pallas as pl
from jax.experimental.pallas import tpu as pltpu

_NEG_SLOPE = 0.2


def _gat_kernel(x_ref, adj_ref, wt_ref, as_ref, ad_ref, b_ref, out_ref,
                ha_ref, ht_ref, asc_ref, asr_ref, adr_ref, d_ref, acc_ref,
                *, block_j, n_nodes):
    j = pl.program_id(0)
    nsteps = pl.num_programs(0)
    f = wt_ref.shape[0]

    @pl.when(j == 0)
    def _prologue():
        h = jax.lax.dot_general(x_ref[...], wt_ref[...],
                                (((1,), (1,)), ((), ())),
                                preferred_element_type=jnp.float32)
        ha_ref[:, :f] = h
        ha_ref[:, f:] = jnp.ones((n_nodes, 1), jnp.float32)
        ht_ref[...] = jnp.transpose(h)
        a_s_row = jax.lax.dot_general(as_ref[...].reshape(1, f), h,
                                      (((1,), (1,)), ((), ())),
                                      preferred_element_type=jnp.float32)
        asr_ref[...] = a_s_row
        asc_ref[...] = jnp.transpose(a_s_row)
        adr_ref[...] = jax.lax.dot_general(ad_ref[...].reshape(1, f), h,
                                           (((1,), (1,)), ((), ())),
                                           preferred_element_type=jnp.float32)
        d_ref[...] = (
            jax.lax.broadcasted_iota(jnp.int32, (block_j, n_nodes), 1)
            - jax.lax.broadcasted_iota(jnp.int32, (block_j, n_nodes), 0))
        acc_ref[...] = jnp.zeros((f + 1, n_nodes), jnp.float32)

    a_s_blk = asc_ref[pl.ds(j * block_j, block_j), :]
    e = a_s_blk + adr_ref[...]
    e = jnp.maximum(e, _NEG_SLOPE * e)
    keep = (adj_ref[...] != 0) & (d_ref[...] != j * block_j)
    p = jnp.where(keep, jnp.exp(e), 0.0)

    ha_blk = ha_ref[pl.ds(j * block_j, block_j), :]
    acc_ref[...] += jax.lax.dot_general(ha_blk, p,
                                        (((0,), (0,)), ((), ())),
                                        preferred_element_type=jnp.float32)

    @pl.when(j == nsteps - 1)
    def _epilogue():
        diag = asr_ref[...] + adr_ref[...]
        diag = jnp.maximum(diag, _NEG_SLOPE * diag)
        p_diag = jnp.exp(diag)
        denom = acc_ref[f:, :] + p_diag
        inv = 1.0 / (denom + 1e-16)
        b_col = b_ref[...].reshape(f, 1)
        out_ref[...] = (acc_ref[:f, :] + ht_ref[...] * p_diag) * inv + b_col


def kernel(x, adj, W, att_src, att_dst, bias):
    n, in_f = x.shape
    f = W.shape[1]

    block_j = 512
    grid = (n // block_j,)

    out_t = pl.pallas_call(
        functools.partial(_gat_kernel, block_j=block_j, n_nodes=n),
        grid=grid,
        in_specs=[
            pl.BlockSpec((n, in_f), lambda j: (0, 0)),
            pl.BlockSpec((block_j, n), lambda j: (j, 0)),
            pl.BlockSpec((f, in_f), lambda j: (0, 0)),
            pl.BlockSpec((1, 1, f), lambda j: (0, 0, 0)),
            pl.BlockSpec((1, 1, f), lambda j: (0, 0, 0)),
            pl.BlockSpec((f,), lambda j: (0,)),
        ],
        out_specs=pl.BlockSpec((f, n), lambda j: (0, 0)),
        out_shape=jax.ShapeDtypeStruct((f, n), jnp.float32),
        scratch_shapes=[
            pltpu.VMEM((n, f + 1), jnp.float32),
            pltpu.VMEM((f, n), jnp.float32),
            pltpu.VMEM((n, 1), jnp.float32),
            pltpu.VMEM((1, n), jnp.float32),
            pltpu.VMEM((1, n), jnp.float32),
            pltpu.VMEM((block_j, n), jnp.int32),
            pltpu.VMEM((f + 1, n), jnp.float32),
        ],
        compiler_params=pltpu.CompilerParams(
            dimension_semantics=("arbitrary",),
        ),
    )(x, adj, W.T, att_src, att_dst, bias)
    return out_t.T
